# baseline (speedup 1.0000x reference)
import functools

import jax
import jax.numpy as jnp
from jax import lax
from jax.experimental import pallas as pl
from jax.experimental.pallas import tpu as pltpu
from jax.experimental.pallas import tpu_sc as plsc

N = 10000
D = 128
FH = 64
E = 320000
G = 64
CLS = 10
NCORE = 2
NSUB = 16
NW = NCORE * NSUB
NPAD = 10240
ROWS_PER_SUB = NPAD // NSUB
CHUNK = 128
EPAD = 327680
NCH_D = EPAD // NW // CHUNK
NCH_E = EPAD // NSUB // CHUNK
RB = 1000
NRB = N // RB

_SC_PARAMS = pltpu.CompilerParams(use_tc_tiling_on_sc=False)


def _vmesh():
    return plsc.VectorSubcoreMesh(core_axis_name="c", subcore_axis_name="s")


def _sc_deg(dst3):
    @functools.partial(
        pl.kernel,
        out_type=jax.ShapeDtypeStruct((NCORE, NPAD, 16), jnp.float32),
        mesh=_vmesh(),
        compiler_params=_SC_PARAMS,
        scratch_types=[
            pltpu.VMEM((NCH_D, CHUNK), jnp.int32),
            pltpu.VMEM((CHUNK, 16), jnp.float32),
            pltpu.VMEM((128, 16), jnp.float32),
            pltpu.VMEM_SHARED((NPAD, 16), jnp.float32),
            pltpu.SemaphoreType.DMA,
        ],
    )
    def k(dst_hbm, out_hbm, dstb, ones, zbuf, accum, sem):
        c = lax.axis_index("c")
        s = lax.axis_index("s")
        w = c * NSUB + s
        row0 = s * ROWS_PER_SUB
        pltpu.async_copy(dst_hbm.at[w], dstb, sem)
        @pl.loop(0, CHUNK)
        def _(i):
            ones[i, :] = jnp.ones((16,), jnp.float32)
        @pl.loop(0, 128)
        def _(i):
            zbuf[i, :] = jnp.zeros((16,), jnp.float32)
        for q in range(5):
            pltpu.async_copy(zbuf, accum.at[pl.ds(row0 + q * 128, 128)], sem)
        pltpu.make_async_copy(dst_hbm.at[w], dstb, sem).wait()
        for q in range(5):
            pltpu.make_async_copy(zbuf, accum.at[pl.ds(row0 + q * 128, 128)],
                                  sem).wait()
        plsc.subcore_barrier()
        @pl.loop(0, NCH_D, step=5)
        def _(j):
            for t in range(5):
                pltpu.async_copy(ones, accum.at[dstb.at[j + t]], sem, add=True)
            for t in range(5):
                pltpu.make_async_copy(ones, accum.at[dstb.at[j + t]], sem).wait()
        plsc.subcore_barrier()
        pltpu.sync_copy(accum.at[pl.ds(row0, ROWS_PER_SUB)],
                        out_hbm.at[c].at[pl.ds(row0, ROWS_PER_SUB)])

    return k(dst3)


def _sc_edge(hs2, src3, dst3):
    @functools.partial(
        pl.kernel,
        out_type=jax.ShapeDtypeStruct((NCORE, NPAD, FH), jnp.float32),
        mesh=_vmesh(),
        compiler_params=_SC_PARAMS,
        scratch_types=[
            pltpu.VMEM((NCH_E, CHUNK), jnp.int32),
            pltpu.VMEM((NCH_E, CHUNK), jnp.int32),
            pltpu.VMEM((5, CHUNK, FH), jnp.float32),
            pltpu.VMEM((128, FH), jnp.float32),
            pltpu.VMEM_SHARED((NPAD, FH), jnp.float32),
            [pltpu.SemaphoreType.DMA] * 5,
            [pltpu.SemaphoreType.DMA] * 5,
        ],
    )
    def k(hs_hbm, src_hbm, dst_hbm, out_hbm,
          srcb, dstb, rows, zbuf, accum, gsem, ssem):
        c = lax.axis_index("c")
        s = lax.axis_index("s")
        row0 = s * ROWS_PER_SUB
        pltpu.async_copy(src_hbm.at[s], srcb, gsem[0])
        pltpu.async_copy(dst_hbm.at[s], dstb, gsem[1])
        @pl.loop(0, 128)
        def _(i):
            for seg in range(FH // 16):
                zbuf[i, pl.ds(seg * 16, 16)] = jnp.zeros((16,), jnp.float32)
        for q in range(5):
            pltpu.async_copy(zbuf, accum.at[pl.ds(row0 + q * 128, 128)],
                             ssem[q])
        pltpu.make_async_copy(src_hbm.at[s], srcb, gsem[0]).wait()
        pltpu.make_async_copy(dst_hbm.at[s], dstb, gsem[1]).wait()
        @pl.loop(0, NCH_E)
        def _(j):
            for seg in range(CHUNK // 16):
                sl = pl.ds(seg * 16, 16)
                srcb[j, sl] = srcb[j, sl] * 2 + c
        for q in range(5):
            pltpu.make_async_copy(zbuf, accum.at[pl.ds(row0 + q * 128, 128)],
                                  ssem[q]).wait()
        plsc.subcore_barrier()
        hsrc = hs_hbm

        def g_fire(n, t):
            pltpu.async_copy(hsrc.at[srcb.at[n]], rows.at[t], gsem[t])

        def g_wait(n, t):
            pltpu.make_async_copy(hsrc.at[srcb.at[n]], rows.at[t],
                                  gsem[t]).wait()

        def s_fire(n, t):
            pltpu.async_copy(rows.at[t], accum.at[dstb.at[n]], ssem[t],
                             add=True)

        def s_wait(n, t):
            pltpu.make_async_copy(rows.at[t], accum.at[dstb.at[n]],
                                  ssem[t]).wait()

        for t in range(5):
            g_fire(t, t)

        @pl.loop(0, NCH_E - 5, step=5)
        def _(j):
            for t in range(5):
                g_wait(j + t, t)
                s_fire(j + t, t)
            for t in range(5):
                s_wait(j + t, t)
                g_fire(j + 5 + t, t)

        for t in range(5):
            g_wait(NCH_E - 5 + t, t)
            s_fire(NCH_E - 5 + t, t)
        for t in range(5):
            s_wait(NCH_E - 5 + t, t)

        plsc.subcore_barrier()
        pltpu.sync_copy(accum.at[pl.ds(row0, ROWS_PER_SUB)],
                        out_hbm.at[c].at[pl.ds(row0, ROWS_PER_SUB)])

    return k(hs2, src3, dst3)


def _tc_head(degp, x, w):
    def body(dp_ref, x_ref, w_ref, hs_ref):
        deg = dp_ref[0, :, 0:1] + dp_ref[1, :, 0:1] + 1.0
        dinv = lax.rsqrt(deg)
        hs_ref[...] = jnp.dot(x_ref[...], w_ref[...],
                              preferred_element_type=jnp.float32) * dinv

    return pl.pallas_call(
        body,
        grid=(NRB,),
        in_specs=[pl.BlockSpec((NCORE, RB, 16), lambda i: (0, i, 0)),
                  pl.BlockSpec((RB, D), lambda i: (i, 0)),
                  pl.BlockSpec((D, D), lambda i: (0, 0))],
        out_specs=pl.BlockSpec((RB, D), lambda i: (i, 0)),
        out_shape=jax.ShapeDtypeStruct((N, D), jnp.float32),
    )(degp, x, w)


def _tc_layer(p, hss, degp, b, w, batch2, wout, bout):
    def body(p_ref, hs_ref, dp_ref, b_ref, w_ref, bt_ref, wo_ref, bo_ref,
             o_ref, cls_ref, ps_ref, pc_ref):
        i = pl.program_id(0)
        deg = dp_ref[0, :, 0:1] + dp_ref[1, :, 0:1] + 1.0
        dinv = lax.rsqrt(deg)
        agg = jnp.concatenate([p_ref[0], p_ref[1]], axis=1) + hs_ref[...]
        x2 = jnp.maximum(dinv * agg + b_ref[...], 0.0)
        o_ref[...] = jnp.dot(x2, w_ref[...],
                             preferred_element_type=jnp.float32) * dinv
        gids = lax.broadcasted_iota(jnp.int32, (G, RB), 0)
        mask = (bt_ref[0] == gids).astype(jnp.float32)
        part = lax.dot_general(mask, x2, (((1,), (0,)), ((), ())),
                               preferred_element_type=jnp.float32)

        @pl.when(i == 0)
        def _():
            ps_ref[...] = jnp.zeros((G, D), jnp.float32)
            pc_ref[...] = jnp.zeros((G, 1), jnp.float32)

        ps_ref[...] += part
        pc_ref[...] += jnp.sum(mask, axis=1, keepdims=True)

        @pl.when(i == NRB - 1)
        def _():
            pooled = ps_ref[...] / jnp.maximum(pc_ref[...], 1.0)
            cls_ref[...] = jnp.dot(pooled, wo_ref[...],
                                   preferred_element_type=jnp.float32) + bo_ref[...]

    return pl.pallas_call(
        body,
        grid=(NRB,),
        in_specs=[pl.BlockSpec((NCORE, RB, FH), lambda i: (0, i, 0)),
                  pl.BlockSpec((RB, D), lambda i: (i, 0)),
                  pl.BlockSpec((NCORE, RB, 16), lambda i: (0, i, 0)),
                  pl.BlockSpec((1, D), lambda i: (0, 0)),
                  pl.BlockSpec((D, D), lambda i: (0, 0)),
                  pl.BlockSpec((1, 1, RB), lambda i: (i, 0, 0)),
                  pl.BlockSpec((D, CLS), lambda i: (0, 0)),
                  pl.BlockSpec((1, CLS), lambda i: (0, 0))],
        out_specs=[pl.BlockSpec((RB, D), lambda i: (i, 0)),
                   pl.BlockSpec((G, CLS), lambda i: (0, 0))],
        out_shape=[jax.ShapeDtypeStruct((N, D), jnp.float32),
                   jax.ShapeDtypeStruct((G, CLS), jnp.float32)],
        scratch_shapes=[pltpu.VMEM((G, D), jnp.float32),
                        pltpu.VMEM((G, 1), jnp.float32)],
    )(p, hss, degp, b, w, batch2, wout, bout)


def kernel(x, edge_index, batch, W1, b1, W2, b2, Wout, bout):
    src = edge_index[0].astype(jnp.int32)
    dst = edge_index[1].astype(jnp.int32)
    npad_edges = EPAD - E
    src = jnp.concatenate([src, jnp.zeros((npad_edges,), jnp.int32)])
    dst = jnp.concatenate([dst, jnp.full((npad_edges,), N + 16, jnp.int32)])
    srcE = src.reshape(NSUB, NCH_E, CHUNK)
    dstE = dst.reshape(NSUB, NCH_E, CHUNK)
    dstD = dst.reshape(NW, NCH_D, CHUNK)
    batch2 = batch.astype(jnp.int32).reshape(NRB, 1, RB)
    boutr = bout.reshape(1, CLS)

    degp = _sc_deg(dstD)
    hss1 = _tc_head(degp, x, W1)

    ws = jnp.stack([W2, jnp.eye(D, dtype=jnp.float32)])
    bs = jnp.stack([b1.reshape(1, D), b2.reshape(1, D)])

    def step(hs, wb):
        w, b = wb
        p = _sc_edge(hs.reshape(2 * N, FH), srcE, dstE)
        nxt, cls = _tc_layer(p, hs, degp, b, w, batch2, Wout, boutr)
        return nxt, cls

    _, clss = lax.scan(step, hss1, (ws, bs))
    return clss[1]


# speedup vs baseline: 2.5196x; 2.5196x over previous
import functools

import jax
import jax.numpy as jnp
from jax import lax
from jax.experimental import pallas as pl
from jax.experimental.pallas import tpu as pltpu
from jax.experimental.pallas import tpu_sc as plsc

N = 10000
D = 128
FH = 64
E = 320000
G = 64
CLS = 10
NCORE = 2
NSUB = 16
NW = NCORE * NSUB
NPAD = 10240
ROWS_PER_SUB = NPAD // NSUB
CHUNK = 125
NCH_D = E // NW // CHUNK
NCH_E = E // NSUB // CHUNK
RB = 2000
NRB = N // RB

_SC_PARAMS = pltpu.CompilerParams(use_tc_tiling_on_sc=False)


def _vmesh():
    return plsc.VectorSubcoreMesh(core_axis_name="c", subcore_axis_name="s")


def _sc_deg(dst3):
    @functools.partial(
        pl.kernel,
        out_type=jax.ShapeDtypeStruct((NCORE, NPAD, 16), jnp.float32),
        mesh=_vmesh(),
        compiler_params=_SC_PARAMS,
        scratch_types=[
            pltpu.VMEM((NCH_D, CHUNK), jnp.int32),
            pltpu.VMEM((CHUNK, 16), jnp.float32),
            pltpu.VMEM((128, 16), jnp.float32),
            pltpu.VMEM_SHARED((NPAD, 16), jnp.float32),
            pltpu.SemaphoreType.DMA,
        ],
    )
    def k(dst_hbm, out_hbm, dstb, ones, zbuf, accum, sem):
        c = lax.axis_index("c")
        s = lax.axis_index("s")
        w = c * NSUB + s
        row0 = s * ROWS_PER_SUB
        pltpu.async_copy(dst_hbm.at[w], dstb, sem)
        @pl.loop(0, CHUNK)
        def _(i):
            ones[i, :] = jnp.ones((16,), jnp.float32)
        @pl.loop(0, 128)
        def _(i):
            zbuf[i, :] = jnp.zeros((16,), jnp.float32)
        for q in range(5):
            pltpu.async_copy(zbuf, accum.at[pl.ds(row0 + q * 128, 128)], sem)
        pltpu.make_async_copy(dst_hbm.at[w], dstb, sem).wait()
        for q in range(5):
            pltpu.make_async_copy(zbuf, accum.at[pl.ds(row0 + q * 128, 128)],
                                  sem).wait()
        plsc.subcore_barrier()
        @pl.loop(0, NCH_D, step=5)
        def _(j):
            for t in range(5):
                pltpu.async_copy(ones, accum.at[dstb.at[j + t]], sem, add=True)
            for t in range(5):
                pltpu.make_async_copy(ones, accum.at[dstb.at[j + t]], sem).wait()
        plsc.subcore_barrier()
        pltpu.sync_copy(accum.at[pl.ds(row0, ROWS_PER_SUB)],
                        out_hbm.at[c].at[pl.ds(row0, ROWS_PER_SUB)])

    return k(dst3)


def _sc_edge(hss, src3, dst3):
    @functools.partial(
        pl.kernel,
        out_type=jax.ShapeDtypeStruct((NCORE, NPAD, FH), jnp.float32),
        mesh=_vmesh(),
        compiler_params=_SC_PARAMS,
        scratch_types=[
            pltpu.VMEM((NCH_E, CHUNK), jnp.int32),
            pltpu.VMEM((NCH_E, CHUNK), jnp.int32),
            pltpu.VMEM((5, CHUNK, FH), jnp.float32),
            pltpu.VMEM((128, FH), jnp.float32),
            pltpu.VMEM_SHARED((NPAD, FH), jnp.float32),
            [pltpu.SemaphoreType.DMA] * 5,
            [pltpu.SemaphoreType.DMA] * 5,
        ],
    )
    def k(hs_hbm, src_hbm, dst_hbm, out_hbm,
          srcb, dstb, rows, zbuf, accum, gsem, ssem):
        c = lax.axis_index("c")
        s = lax.axis_index("s")
        row0 = s * ROWS_PER_SUB
        pltpu.async_copy(src_hbm.at[s], srcb, gsem[0])
        pltpu.async_copy(dst_hbm.at[s], dstb, gsem[1])
        @pl.loop(0, 128)
        def _(i):
            for seg in range(FH // 16):
                zbuf[i, pl.ds(seg * 16, 16)] = jnp.zeros((16,), jnp.float32)
        for q in range(5):
            pltpu.async_copy(zbuf, accum.at[pl.ds(row0 + q * 128, 128)],
                             ssem[q])
        pltpu.make_async_copy(src_hbm.at[s], srcb, gsem[0]).wait()
        pltpu.make_async_copy(dst_hbm.at[s], dstb, gsem[1]).wait()
        for q in range(5):
            pltpu.make_async_copy(zbuf, accum.at[pl.ds(row0 + q * 128, 128)],
                                  ssem[q]).wait()
        plsc.subcore_barrier()
        hsrc = hs_hbm.at[c]

        def g_fire(n, t):
            pltpu.async_copy(hsrc.at[srcb.at[n]], rows.at[t], gsem[t])

        def g_wait(n, t):
            pltpu.make_async_copy(hsrc.at[srcb.at[n]], rows.at[t],
                                  gsem[t]).wait()

        def s_fire(n, t):
            pltpu.async_copy(rows.at[t], accum.at[dstb.at[n]], ssem[t],
                             add=True)

        def s_wait(n, t):
            pltpu.make_async_copy(rows.at[t], accum.at[dstb.at[n]],
                                  ssem[t]).wait()

        for t in range(5):
            g_fire(t, t)

        @pl.loop(0, NCH_E - 5, step=5)
        def _(j):
            for t in range(5):
                g_wait(j + t, t)
                s_fire(j + t, t)
            for t in range(5):
                s_wait(j + t, t)
                g_fire(j + 5 + t, t)

        for t in range(5):
            g_wait(NCH_E - 5 + t, t)
            s_fire(NCH_E - 5 + t, t)
        for t in range(5):
            s_wait(NCH_E - 5 + t, t)

        plsc.subcore_barrier()
        pltpu.sync_copy(accum.at[pl.ds(row0, ROWS_PER_SUB)],
                        out_hbm.at[c].at[pl.ds(row0, ROWS_PER_SUB)])

    return k(hss, src3, dst3)


def _tc_head(degp, x, w):
    def body(dp_ref, x_ref, w_ref, hs_ref):
        deg = dp_ref[0, :, 0:1] + dp_ref[1, :, 0:1] + 1.0
        dinv = lax.rsqrt(deg)
        hs = jnp.dot(x_ref[...], w_ref[...],
                     preferred_element_type=jnp.float32) * dinv
        hs_ref[0] = hs[:, :FH]
        hs_ref[1] = hs[:, FH:]

    return pl.pallas_call(
        body,
        grid=(NRB,),
        in_specs=[pl.BlockSpec((NCORE, RB, 16), lambda i: (0, i, 0)),
                  pl.BlockSpec((RB, D), lambda i: (i, 0)),
                  pl.BlockSpec((D, D), lambda i: (0, 0))],
        out_specs=pl.BlockSpec((NCORE, RB, FH), lambda i: (0, i, 0)),
        out_shape=jax.ShapeDtypeStruct((NCORE, N, FH), jnp.float32),
    )(degp, x, w)


def _tc_layer(p, hss, degp, b, w, batch2, wout, bout):
    def body(p_ref, hs_ref, dp_ref, b_ref, w_ref, bt_ref, wo_ref, bo_ref,
             o_ref, cls_ref, ps_ref, pc_ref):
        i = pl.program_id(0)
        deg = dp_ref[0, :, 0:1] + dp_ref[1, :, 0:1] + 1.0
        dinv = lax.rsqrt(deg)
        agg = jnp.concatenate([p_ref[0] + hs_ref[0], p_ref[1] + hs_ref[1]],
                              axis=1)
        x2 = jnp.maximum(dinv * agg + b_ref[...], 0.0)
        h2 = jnp.dot(x2, w_ref[...],
                     preferred_element_type=jnp.float32) * dinv
        o_ref[0] = h2[:, :FH]
        o_ref[1] = h2[:, FH:]
        gids = lax.broadcasted_iota(jnp.int32, (G, RB), 0)
        mask = (bt_ref[0] == gids).astype(jnp.float32)
        part = lax.dot_general(mask, x2, (((1,), (0,)), ((), ())),
                               preferred_element_type=jnp.float32)

        @pl.when(i == 0)
        def _():
            ps_ref[...] = jnp.zeros((G, D), jnp.float32)
            pc_ref[...] = jnp.zeros((G, 1), jnp.float32)

        ps_ref[...] += part
        pc_ref[...] += jnp.sum(mask, axis=1, keepdims=True)

        @pl.when(i == NRB - 1)
        def _():
            pooled = ps_ref[...] / jnp.maximum(pc_ref[...], 1.0)
            cls_ref[...] = jnp.dot(pooled, wo_ref[...],
                                   preferred_element_type=jnp.float32) + bo_ref[...]

    return pl.pallas_call(
        body,
        grid=(NRB,),
        in_specs=[pl.BlockSpec((NCORE, RB, FH), lambda i: (0, i, 0)),
                  pl.BlockSpec((NCORE, RB, FH), lambda i: (0, i, 0)),
                  pl.BlockSpec((NCORE, RB, 16), lambda i: (0, i, 0)),
                  pl.BlockSpec((1, D), lambda i: (0, 0)),
                  pl.BlockSpec((D, D), lambda i: (0, 0)),
                  pl.BlockSpec((1, 1, RB), lambda i: (i, 0, 0)),
                  pl.BlockSpec((D, CLS), lambda i: (0, 0)),
                  pl.BlockSpec((1, CLS), lambda i: (0, 0))],
        out_specs=[pl.BlockSpec((NCORE, RB, FH), lambda i: (0, i, 0)),
                   pl.BlockSpec((G, CLS), lambda i: (0, 0))],
        out_shape=[jax.ShapeDtypeStruct((NCORE, N, FH), jnp.float32),
                   jax.ShapeDtypeStruct((G, CLS), jnp.float32)],
        scratch_shapes=[pltpu.VMEM((G, D), jnp.float32),
                        pltpu.VMEM((G, 1), jnp.float32)],
    )(p, hss, degp, b, w, batch2, wout, bout)


def kernel(x, edge_index, batch, W1, b1, W2, b2, Wout, bout):
    src = edge_index[0].astype(jnp.int32)
    dst = edge_index[1].astype(jnp.int32)
    srcE = src.reshape(NSUB, NCH_E, CHUNK)
    dstE = dst.reshape(NSUB, NCH_E, CHUNK)
    dstD = dst.reshape(NW, NCH_D, CHUNK)
    batch2 = batch.astype(jnp.int32).reshape(NRB, 1, RB)
    boutr = bout.reshape(1, CLS)

    degp = _sc_deg(dstD)
    hss1 = _tc_head(degp, x, W1)

    ws = jnp.stack([W2, jnp.eye(D, dtype=jnp.float32)])
    bs = jnp.stack([b1.reshape(1, D), b2.reshape(1, D)])

    def step(hs, wb):
        w, b = wb
        p = _sc_edge(hs, srcE, dstE)
        nxt, cls = _tc_layer(p, hs, degp, b, w, batch2, Wout, boutr)
        return nxt, cls

    _, clss = lax.scan(step, hss1, (ws, bs))
    return clss[1]
